# baseline (speedup 1.0000x reference)
import functools

import jax
import jax.numpy as jnp
from jax.experimental import pallas as pl

SEQ_LEN = 2048
START_TOK = 0
END_TOK = 2
PAD_TOK = 1


def _trim_budgets(L1, L2, budget):
    if L1 + L2 <= budget:
        return L1, L2
    k1 = min(L1, max((budget + 1) // 2, budget - L2))
    k2 = min(L2, max(budget // 2, budget - L1))
    return max(k1, 0), max(k2, 0)


@functools.cache
def _build_packer(B, L1, L2):
    budget = SEQ_LEN - 4
    k1, k2 = _trim_budgets(L1, L2, budget)
    pad = SEQ_LEN - (4 + k1 + k2)
    assert pad == 0

    w1 = -(-k1 // 128) * 128
    w2 = -(-k2 // 128) * 128

    rb = 8
    nsteps = B // rb

    def body(s1_ref, s2_ref, o_ref):
        s1 = s1_ref[:, :k1]
        s2 = s2_ref[:, :k2]
        start = jnp.full((rb, 1), START_TOK, jnp.int32)
        split = jnp.full((rb, 2), END_TOK, jnp.int32)
        end = jnp.full((rb, 1), END_TOK, jnp.int32)
        o_ref[...] = jnp.concatenate([start, s1, split, s2, end], axis=1)

    return pl.pallas_call(
        body,
        grid=(nsteps,),
        in_specs=[
            pl.BlockSpec((rb, w1), lambda i: (i, 0)),
            pl.BlockSpec((rb, w2), lambda i: (i, 0)),
        ],
        out_specs=pl.BlockSpec((rb, SEQ_LEN), lambda i: (i, 0)),
        out_shape=jax.ShapeDtypeStruct((B, SEQ_LEN), jnp.int32),
    )


def kernel(segment_1, segment_2):
    B, L1 = segment_1.shape
    L2 = segment_2.shape[1]
    return _build_packer(B, L1, L2)(segment_1, segment_2)


# speedup vs baseline: 1.0179x; 1.0179x over previous
import functools

import jax
import jax.numpy as jnp
from jax.experimental import pallas as pl

SEQ_LEN = 2048
START_TOK = 0
END_TOK = 2
PAD_TOK = 1


def _trim_budgets(L1, L2, budget):
    if L1 + L2 <= budget:
        return L1, L2
    k1 = min(L1, max((budget + 1) // 2, budget - L2))
    k2 = min(L2, max(budget // 2, budget - L1))
    return max(k1, 0), max(k2, 0)


@functools.cache
def _build_packer(B, L1, L2):
    budget = SEQ_LEN - 4
    k1, k2 = _trim_budgets(L1, L2, budget)
    pad = SEQ_LEN - (4 + k1 + k2)
    assert pad == 0

    w1 = -(-k1 // 128) * 128
    w2 = -(-k2 // 128) * 128

    def body(s1_ref, s2_ref, o_ref):
        s1 = s1_ref[:, :k1]
        s2 = s2_ref[:, :k2]
        start = jnp.full((B, 1), START_TOK, jnp.int32)
        split = jnp.full((B, 2), END_TOK, jnp.int32)
        end = jnp.full((B, 1), END_TOK, jnp.int32)
        o_ref[...] = jnp.concatenate([start, s1, split, s2, end], axis=1)

    return pl.pallas_call(
        body,
        grid=(1,),
        in_specs=[
            pl.BlockSpec((B, w1), lambda i: (0, 0)),
            pl.BlockSpec((B, w2), lambda i: (0, 0)),
        ],
        out_specs=pl.BlockSpec((B, SEQ_LEN), lambda i: (0, 0)),
        out_shape=jax.ShapeDtypeStruct((B, SEQ_LEN), jnp.int32),
    )


def kernel(segment_1, segment_2):
    B, L1 = segment_1.shape
    L2 = segment_2.shape[1]
    return _build_packer(B, L1, L2)(segment_1, segment_2)
